# baseline (speedup 1.0000x reference)
import functools

import jax
import jax.numpy as jnp
from jax import lax
from jax.experimental import pallas as pl
from jax.experimental.pallas import tpu as pltpu
from jax.experimental.pallas import tpu_sc as plsc

_CUT0 = 20000
_CUT1 = 60000
_VOCAB = 100000

_NC = 2
_NS = 16
_NW = _NC * _NS


def _proj_body(x_ref, p1_ref, p2_ref, o1_ref, o2_ref):
    o1_ref[...] = jnp.dot(x_ref[...], p1_ref[...],
                          preferred_element_type=jnp.float32)
    o2_ref[...] = jnp.dot(x_ref[...], p2_ref[...],
                          preferred_element_type=jnp.float32)


def _tail_proj(x, proj1, proj2, rb):
    n, d = x.shape
    p1 = proj1.shape[1]
    p2 = proj2.shape[1]
    return pl.pallas_call(
        _proj_body,
        grid=(n // rb,),
        in_specs=[
            pl.BlockSpec((rb, d), lambda i: (i, 0)),
            pl.BlockSpec((d, p1), lambda i: (0, 0)),
            pl.BlockSpec((d, p2), lambda i: (0, 0)),
        ],
        out_specs=[
            pl.BlockSpec((rb, p1), lambda i: (i, 0)),
            pl.BlockSpec((rb, p2), lambda i: (i, 0)),
        ],
        out_shape=[jax.ShapeDtypeStruct((n, p1), jnp.float32),
                   jax.ShapeDtypeStruct((n, p2), jnp.float32)],
    )(x, proj1, proj2)


def _flash_nll_body(x_ref, proj_ref, w_ref, b_ref, tgt_ref, cnt_ref, out_ref,
                    ph, m, s, t, *, rb, cb, ncb, head):
    j = pl.program_id(0)
    i = pl.program_id(1)
    rows = pl.ds(i * rb, rb)
    active = (i * rb) < cnt_ref[0]

    @pl.when(active)
    def _step():
        @pl.when(j == 0)
        def _init():
            if head:
                ph[rows, :] = jnp.dot(x_ref[...], proj_ref[...],
                                      preferred_element_type=jnp.float32)
            else:
                ph[rows, :] = x_ref[...]
            m[rows, :] = jnp.full((rb, 1), -1e30, jnp.float32)
            s[rows, :] = jnp.zeros((rb, 1), jnp.float32)
            t[rows, :] = jnp.zeros((rb, 1), jnp.float32)

        tcol = tgt_ref[:, :1]
        if head:
            idx = jnp.where(tcol >= _CUT1, _CUT0,
                            jnp.where(tcol >= _CUT0, _CUT0 + 1, tcol))
        else:
            idx = tcol

        logits = jax.lax.dot_general(
            ph[rows, :], w_ref[...], (((1,), (1,)), ((), ())),
            preferred_element_type=jnp.float32)
        logits = logits + b_ref[0, :, :]
        col_ids = j * cb + jax.lax.broadcasted_iota(jnp.int32, logits.shape, 1)

        t[rows, :] += jnp.sum(jnp.where(col_ids == idx, logits, 0.0),
                              axis=1, keepdims=True)
        bm = jnp.max(logits, axis=1, keepdims=True)
        m_new = jnp.maximum(m[rows, :], bm)
        s[rows, :] = (s[rows, :] * jnp.exp(m[rows, :] - m_new)
                      + jnp.sum(jnp.exp(logits - m_new), axis=1,
                                keepdims=True))
        m[rows, :] = m_new

        @pl.when(j == ncb - 1)
        def _finish():
            out_ref[rows, :] = (m[rows, :] + jnp.log(s[rows, :])) - t[rows, :]


def _cluster_nll(x, proj, wp, bp, tgtb, cnt, *, cb, ncb, head, rb):
    n, d = x.shape
    p = proj.shape[1] if head else d
    nrb = n // rb

    body = functools.partial(_flash_nll_body, rb=rb, cb=cb, ncb=ncb,
                             head=head)
    out = pl.pallas_call(
        body,
        grid=(ncb, nrb),
        in_specs=[
            pl.BlockSpec((rb, d),
                         lambda j, i: (jnp.where(j == 0, i, 0), 0)),
            pl.BlockSpec((d, proj.shape[1]), lambda j, i: (0, 0)),
            pl.BlockSpec((cb, p), lambda j, i: (j, 0)),
            pl.BlockSpec((1, 1, cb), lambda j, i: (j, 0, 0)),
            pl.BlockSpec((rb, 128), lambda j, i: (i, 0)),
            pl.BlockSpec(memory_space=pltpu.SMEM),
        ],
        out_specs=pl.BlockSpec((n, 1), lambda j, i: (0, 0)),
        out_shape=jax.ShapeDtypeStruct((n, 1), jnp.float32),
        scratch_shapes=[
            pltpu.VMEM((n, p), jnp.float32),
            pltpu.VMEM((n, 1), jnp.float32),
            pltpu.VMEM((n, 1), jnp.float32),
            pltpu.VMEM((n, 1), jnp.float32),
        ],
        compiler_params=pltpu.CompilerParams(
            vmem_limit_bytes=100 * 1024 * 1024),
    )(x, proj, wp, bp, tgtb, cnt)
    return out


def _sc_gather(ph1, ph2, tgtf, ix1, ix2):
    n, d1 = ph1.shape
    d2 = ph2.shape[1]
    spw = n // _NW
    chunk = 128
    nch = spw // chunk
    mesh = plsc.VectorSubcoreMesh(core_axis_name="c", subcore_axis_name="s")

    @functools.partial(
        pl.kernel, mesh=mesh,
        out_type=[jax.ShapeDtypeStruct((n, d1), jnp.float32),
                  jax.ShapeDtypeStruct((n, d2), jnp.float32),
                  jax.ShapeDtypeStruct((n, 128), jnp.float32),
                  jax.ShapeDtypeStruct((n, 128), jnp.float32)],
        scratch_types=[
            pltpu.VMEM((spw,), jnp.int32),
            pltpu.VMEM((chunk, d1), jnp.float32),
            pltpu.VMEM((chunk, d1), jnp.float32),
            pltpu.VMEM((chunk, 128), jnp.float32),
            pltpu.VMEM((chunk, 128), jnp.float32),
            pltpu.SemaphoreType.DMA,
            pltpu.SemaphoreType.DMA,
            pltpu.SemaphoreType.DMA,
            pltpu.SemaphoreType.DMA,
        ],
    )
    def k(ph1_hbm, ph2_hbm, tgt_hbm, ix1_hbm, ix2_hbm,
          o1_hbm, o2_hbm, g1_hbm, g2_hbm,
          idx_v, a1_v, b1_v, a2_v, b2_v, sa, sb, sc, sd):
        wid = lax.axis_index("s") * _NC + lax.axis_index("c")
        base = wid * spw

        def fire(tab_hbm, A, B, s1, s2):
            cp0 = pltpu.async_copy(
                tab_hbm.at[idx_v.at[pl.ds(0, chunk)]], A, s1)
            cp1 = pltpu.async_copy(
                tab_hbm.at[idx_v.at[pl.ds(chunk, chunk)]], B, s2)
            return cp0, cp1

        def drain(cps, A, B, out_hbm):
            cps[0].wait()
            pltpu.sync_copy(A, out_hbm.at[pl.ds(base, chunk)])
            cps[1].wait()
            pltpu.sync_copy(B, out_hbm.at[pl.ds(base + chunk, chunk)])

        pltpu.sync_copy(ix1_hbm.at[pl.ds(base, spw)], idx_v)
        cps = fire(ph1_hbm, a1_v, b1_v, sa, sb)
        gps = fire(tgt_hbm, a2_v, b2_v, sc, sd)
        drain(cps, a1_v, b1_v, o1_hbm)
        drain(gps, a2_v, b2_v, g1_hbm)
        pltpu.sync_copy(ix2_hbm.at[pl.ds(base, spw)], idx_v)
        cps = fire(ph2_hbm, a2_v, b2_v, sa, sb)
        drain(cps, a2_v, b2_v, o2_hbm)
        gps = fire(tgt_hbm, a2_v, b2_v, sc, sd)
        drain(gps, a2_v, b2_v, g2_hbm)

    return k(ph1, ph2, tgtf, ix1, ix2)


def _sc_nll_gather(nllt, pos):
    n2, w = nllt.shape
    n = pos.shape[0]
    spw = n // _NW
    chunk = 128
    nch = spw // chunk
    mesh = plsc.VectorSubcoreMesh(core_axis_name="c", subcore_axis_name="s")

    @functools.partial(
        pl.kernel, mesh=mesh,
        out_type=jax.ShapeDtypeStruct((n, w), jnp.float32),
        scratch_types=[
            pltpu.VMEM((spw,), jnp.int32),
            pltpu.VMEM((chunk, w), jnp.float32),
            pltpu.VMEM((chunk, w), jnp.float32),
            pltpu.SemaphoreType.DMA,
            pltpu.SemaphoreType.DMA,
        ],
    )
    def k(tab_hbm, pos_hbm, out_hbm, pos_v, a_v, b_v, sa, sb):
        wid = lax.axis_index("s") * _NC + lax.axis_index("c")
        base = wid * spw
        pltpu.sync_copy(pos_hbm.at[pl.ds(base, spw)], pos_v)
        bufs = (a_v, b_v)
        sems = (sa, sb)
        cps = []
        for c in range(nch):
            cps.append(pltpu.async_copy(
                tab_hbm.at[pos_v.at[pl.ds(c * chunk, chunk)]],
                bufs[c % 2], sems[c % 2]))
        for c in range(nch):
            cps[c].wait()
            pltpu.sync_copy(
                bufs[c % 2], out_hbm.at[pl.ds(base + c * chunk, chunk)])

    return k(nllt, pos)


def _combine_body(h_ref, g_ref, cl_ref, out_ref):
    cl = cl_ref[...]
    out_ref[...] = h_ref[...] + jnp.where(cl > 0, g_ref[...], 0.0)


def _combine(head2, g, clid2):
    n = head2.shape[0]
    return pl.pallas_call(
        _combine_body,
        out_shape=jax.ShapeDtypeStruct((n, 16), jnp.float32),
    )(head2, g, clid2)


def _pad_wb(w, b, cb):
    nv = w.shape[0]
    ncb = pl.cdiv(nv, cb)
    npad = ncb * cb - nv
    wp = jnp.concatenate([w, jnp.zeros((npad, w.shape[1]), w.dtype)], axis=0)
    bp = jnp.full((ncb * cb,), -1e30, jnp.float32).at[:nv].set(b)
    return wp, bp.reshape(ncb, 1, cb), ncb


def kernel(input, target, cluster_weight, cluster_bias, proj0, proj1, proj2,
           w0, b0, w1, b1, w2, b2):
    n = input.shape[0]
    rb = 256
    tgt = target.astype(jnp.int32)
    rows = jnp.arange(n, dtype=jnp.int32)

    m1 = (tgt >= _CUT0) & (tgt < _CUT1)
    m2 = tgt >= _CUT1
    pos1 = jnp.cumsum(m1.astype(jnp.int32)) - 1
    pos2 = jnp.cumsum(m2.astype(jnp.int32)) - 1
    cnt1 = jnp.sum(m1.astype(jnp.int32))
    cnt2 = jnp.sum(m2.astype(jnp.int32))
    ix1 = jnp.zeros((n,), jnp.int32).at[
        jnp.where(m1, pos1, n)].set(rows, mode="drop")
    ix2 = jnp.zeros((n,), jnp.int32).at[
        jnp.where(m2, pos2, n)].set(rows, mode="drop")
    slot = rows
    pos = jnp.where(m1, pos1, jnp.where(m2, pos2 + n, 0))
    clid = m1.astype(jnp.int32) + 2 * m2.astype(jnp.int32)

    d = input.shape[1]
    proj2p = jnp.concatenate(
        [proj2, jnp.zeros((d, 128 - proj2.shape[1]), jnp.float32)], axis=1)
    ph1, ph2 = _tail_proj(input, proj1, proj2p, rb)
    tgtf = jnp.broadcast_to(
        jax.lax.bitcast_convert_type(tgt, jnp.float32)[:, None], (n, 128))
    phc1, phc2, gt1f, gt2f = _sc_gather(ph1, ph2, tgtf, ix1, ix2)
    gt1 = jax.lax.bitcast_convert_type(gt1f, jnp.int32)
    gt2 = jax.lax.bitcast_convert_type(gt2f, jnp.int32)
    ct1b = jnp.where(slot[:, None] < cnt1, gt1 - _CUT0, -1)
    ct2b = jnp.where(slot[:, None] < cnt2, gt2 - _CUT1, -1)

    hw, hb, ncb_h = _pad_wb(jnp.concatenate([w0, cluster_weight], axis=0),
                            jnp.concatenate([b0, cluster_bias], axis=0), 1024)
    w1p, b1p, ncb_1 = _pad_wb(w1, b1, 2048)
    w2c = jnp.concatenate(
        [w2, jnp.zeros((w2.shape[0], 128 - w2.shape[1]), jnp.float32)],
        axis=1)
    w2p, b2p, ncb_2 = _pad_wb(w2c, b2, 2048)

    tgtb = jnp.broadcast_to(tgt[:, None], (n, 128))
    nfull = jnp.full((1,), n, jnp.int32)

    head = _cluster_nll(input, proj0, hw, hb, tgtb, nfull,
                        cb=1024, ncb=ncb_h, head=True, rb=rb)
    t1c = _cluster_nll(phc1, proj1, w1p, b1p, ct1b, cnt1.reshape(1),
                       cb=2048, ncb=ncb_1, head=False, rb=rb)
    t2c = _cluster_nll(phc2, proj2p, w2p, b2p, ct2b, cnt2.reshape(1),
                       cb=2048, ncb=ncb_2, head=False, rb=rb)

    nllt = jnp.broadcast_to(jnp.concatenate([t1c, t2c], axis=0), (2 * n, 128))
    g = _sc_nll_gather(nllt, pos)

    head2 = jnp.broadcast_to(head, (n, 16))
    clid2 = jnp.broadcast_to(clid[:, None], (n, 16))
    return _combine(head2, g[:, :16], clid2)[:, 0]


# speedup vs baseline: 1.0801x; 1.0801x over previous
import functools

import jax
import jax.numpy as jnp
from jax import lax
from jax.experimental import pallas as pl
from jax.experimental.pallas import tpu as pltpu
from jax.experimental.pallas import tpu_sc as plsc

_CUT0 = 20000
_CUT1 = 60000
_VOCAB = 100000

_NC = 2
_NS = 16
_NW = _NC * _NS


def _proj_body(x_ref, p1_ref, p2_ref, o1_ref, o2_ref):
    o1_ref[...] = jnp.dot(x_ref[...], p1_ref[...],
                          preferred_element_type=jnp.float32)
    o2_ref[...] = jnp.dot(x_ref[...], p2_ref[...],
                          preferred_element_type=jnp.float32)


def _tail_proj(x, proj1, proj2, rb):
    n, d = x.shape
    p1 = proj1.shape[1]
    p2 = proj2.shape[1]
    return pl.pallas_call(
        _proj_body,
        grid=(n // rb,),
        in_specs=[
            pl.BlockSpec((rb, d), lambda i: (i, 0)),
            pl.BlockSpec((d, p1), lambda i: (0, 0)),
            pl.BlockSpec((d, p2), lambda i: (0, 0)),
        ],
        out_specs=[
            pl.BlockSpec((rb, p1), lambda i: (i, 0)),
            pl.BlockSpec((rb, p2), lambda i: (i, 0)),
        ],
        out_shape=[jax.ShapeDtypeStruct((n, p1), jnp.float32),
                   jax.ShapeDtypeStruct((n, p2), jnp.float32)],
    )(x, proj1, proj2)


def _flash_nll_body(x_ref, proj_ref, w_ref, b_ref, tgt_ref, cnt_ref, out_ref,
                    ph, m, s, t, *, rb, cb, ncb, head):
    j = pl.program_id(0)
    i = pl.program_id(1)
    rows = pl.ds(i * rb, rb)
    active = (i * rb) < cnt_ref[0]

    @pl.when(active)
    def _step():
        @pl.when(j == 0)
        def _init():
            if head:
                ph[rows, :] = jnp.dot(x_ref[...], proj_ref[...],
                                      preferred_element_type=jnp.float32)
            else:
                ph[rows, :] = x_ref[...]
            m[rows, :] = jnp.full((rb, 1), -1e30, jnp.float32)
            s[rows, :] = jnp.zeros((rb, 1), jnp.float32)
            t[rows, :] = jnp.zeros((rb, 1), jnp.float32)

        tcol = tgt_ref[:, :1]
        if head:
            idx = jnp.where(tcol >= _CUT1, _CUT0,
                            jnp.where(tcol >= _CUT0, _CUT0 + 1, tcol))
        else:
            idx = tcol

        logits = jax.lax.dot_general(
            ph[rows, :], w_ref[...], (((1,), (1,)), ((), ())),
            preferred_element_type=jnp.float32)
        logits = logits + b_ref[0, :, :]
        col_ids = j * cb + jax.lax.broadcasted_iota(jnp.int32, logits.shape, 1)

        t[rows, :] += jnp.sum(jnp.where(col_ids == idx, logits, 0.0),
                              axis=1, keepdims=True)
        bm = jnp.max(logits, axis=1, keepdims=True)
        m_new = jnp.maximum(m[rows, :], bm)
        s[rows, :] = (s[rows, :] * jnp.exp(m[rows, :] - m_new)
                      + jnp.sum(jnp.exp(logits - m_new), axis=1,
                                keepdims=True))
        m[rows, :] = m_new

        @pl.when(j == ncb - 1)
        def _finish():
            out_ref[rows, :] = (m[rows, :] + jnp.log(s[rows, :])) - t[rows, :]


def _cluster_nll(x, proj, wp, bp, tgtb, cnt, *, cb, ncb, head, rb):
    n, d = x.shape
    p = proj.shape[1] if head else d
    nrb = n // rb

    body = functools.partial(_flash_nll_body, rb=rb, cb=cb, ncb=ncb,
                             head=head)
    out = pl.pallas_call(
        body,
        grid=(ncb, nrb),
        in_specs=[
            pl.BlockSpec((rb, d),
                         lambda j, i: (jnp.where(j == 0, i, 0), 0)),
            pl.BlockSpec((d, proj.shape[1]), lambda j, i: (0, 0)),
            pl.BlockSpec((cb, p), lambda j, i: (j, 0)),
            pl.BlockSpec((1, 1, cb), lambda j, i: (j, 0, 0)),
            pl.BlockSpec((rb, 128), lambda j, i: (i, 0)),
            pl.BlockSpec(memory_space=pltpu.SMEM),
        ],
        out_specs=pl.BlockSpec((n, 1), lambda j, i: (0, 0)),
        out_shape=jax.ShapeDtypeStruct((n, 1), jnp.float32),
        scratch_shapes=[
            pltpu.VMEM((n, p), jnp.float32),
            pltpu.VMEM((n, 1), jnp.float32),
            pltpu.VMEM((n, 1), jnp.float32),
            pltpu.VMEM((n, 1), jnp.float32),
        ],
        compiler_params=pltpu.CompilerParams(
            vmem_limit_bytes=100 * 1024 * 1024),
    )(x, proj, wp, bp, tgtb, cnt)
    return out


def _sc_gather(ph1, ph2, ix1, ix2):
    n, d1 = ph1.shape
    d2 = ph2.shape[1]
    spw = n // _NW
    chunk = 128
    nch = spw // chunk
    mesh = plsc.VectorSubcoreMesh(core_axis_name="c", subcore_axis_name="s")

    @functools.partial(
        pl.kernel, mesh=mesh,
        out_type=[jax.ShapeDtypeStruct((n, d1), jnp.float32),
                  jax.ShapeDtypeStruct((n, d2), jnp.float32)],
        scratch_types=[
            pltpu.VMEM((spw,), jnp.int32),
            pltpu.VMEM((chunk, d1), jnp.float32),
            pltpu.VMEM((chunk, d1), jnp.float32),
            pltpu.VMEM((chunk, d2), jnp.float32),
            pltpu.VMEM((chunk, d2), jnp.float32),
            pltpu.SemaphoreType.DMA,
            pltpu.SemaphoreType.DMA,
        ],
    )
    def k(ph1_hbm, ph2_hbm, ix1_hbm, ix2_hbm, o1_hbm, o2_hbm,
          idx_v, a1_v, b1_v, a2_v, b2_v, sa, sb):
        wid = lax.axis_index("s") * _NC + lax.axis_index("c")
        base = wid * spw
        for (tab_hbm, ix_hbm, out_hbm, a_v, b_v) in (
                (ph1_hbm, ix1_hbm, o1_hbm, a1_v, b1_v),
                (ph2_hbm, ix2_hbm, o2_hbm, a2_v, b2_v)):
            pltpu.sync_copy(ix_hbm.at[pl.ds(base, spw)], idx_v)
            cp0 = pltpu.async_copy(
                tab_hbm.at[idx_v.at[pl.ds(0, chunk)]], a_v, sa)
            cp1 = pltpu.async_copy(
                tab_hbm.at[idx_v.at[pl.ds(chunk, chunk)]], b_v, sb)
            cp0.wait()
            pltpu.sync_copy(a_v, out_hbm.at[pl.ds(base, chunk)])
            cp1.wait()
            pltpu.sync_copy(b_v, out_hbm.at[pl.ds(base + chunk, chunk)])

    return k(ph1, ph2, ix1, ix2)


def _sc_nll_gather(nllt, pos):
    n2, w = nllt.shape
    n = pos.shape[0]
    spw = n // _NW
    chunk = 128
    nch = spw // chunk
    mesh = plsc.VectorSubcoreMesh(core_axis_name="c", subcore_axis_name="s")

    @functools.partial(
        pl.kernel, mesh=mesh,
        out_type=jax.ShapeDtypeStruct((n, w), jnp.float32),
        scratch_types=[
            pltpu.VMEM((spw,), jnp.int32),
            pltpu.VMEM((chunk, w), jnp.float32),
            pltpu.VMEM((chunk, w), jnp.float32),
            pltpu.SemaphoreType.DMA,
            pltpu.SemaphoreType.DMA,
        ],
    )
    def k(tab_hbm, pos_hbm, out_hbm, pos_v, a_v, b_v, sa, sb):
        wid = lax.axis_index("s") * _NC + lax.axis_index("c")
        base = wid * spw
        pltpu.sync_copy(pos_hbm.at[pl.ds(base, spw)], pos_v)
        bufs = (a_v, b_v)
        sems = (sa, sb)
        cps = []
        for c in range(nch):
            cps.append(pltpu.async_copy(
                tab_hbm.at[pos_v.at[pl.ds(c * chunk, chunk)]],
                bufs[c % 2], sems[c % 2]))
        for c in range(nch):
            cps[c].wait()
            pltpu.sync_copy(
                bufs[c % 2], out_hbm.at[pl.ds(base + c * chunk, chunk)])

    return k(nllt, pos)


def _combine_body(h_ref, g_ref, cl_ref, out_ref):
    cl = cl_ref[...]
    out_ref[...] = h_ref[...] + jnp.where(cl > 0, g_ref[...], 0.0)


def _combine(head2, g, clid2):
    n = head2.shape[0]
    return pl.pallas_call(
        _combine_body,
        out_shape=jax.ShapeDtypeStruct((n, 16), jnp.float32),
    )(head2, g, clid2)


def _pad_wb(w, b, cb):
    nv = w.shape[0]
    ncb = pl.cdiv(nv, cb)
    npad = ncb * cb - nv
    wp = jnp.concatenate([w, jnp.zeros((npad, w.shape[1]), w.dtype)], axis=0)
    bp = jnp.full((ncb * cb,), -1e30, jnp.float32).at[:nv].set(b)
    return wp, bp.reshape(ncb, 1, cb), ncb


def kernel(input, target, cluster_weight, cluster_bias, proj0, proj1, proj2,
           w0, b0, w1, b1, w2, b2):
    n = input.shape[0]
    rb = 256
    tgt = target.astype(jnp.int32)
    rows = jnp.arange(n, dtype=jnp.int32)

    m1 = (tgt >= _CUT0) & (tgt < _CUT1)
    m2 = tgt >= _CUT1
    pos1 = jnp.cumsum(m1.astype(jnp.int32)) - 1
    pos2 = jnp.cumsum(m2.astype(jnp.int32)) - 1
    cnt1 = jnp.sum(m1.astype(jnp.int32))
    cnt2 = jnp.sum(m2.astype(jnp.int32))
    ix1 = jnp.zeros((n,), jnp.int32).at[
        jnp.where(m1, pos1, n)].set(rows, mode="drop")
    ix2 = jnp.zeros((n,), jnp.int32).at[
        jnp.where(m2, pos2, n)].set(rows, mode="drop")
    slot = rows
    pos = jnp.where(m1, pos1, jnp.where(m2, pos2 + n, 0))
    clid = m1.astype(jnp.int32) + 2 * m2.astype(jnp.int32)

    d = input.shape[1]
    proj2p = jnp.concatenate(
        [proj2, jnp.zeros((d, 128 - proj2.shape[1]), jnp.float32)], axis=1)
    ph1, ph2 = _tail_proj(input, proj1, proj2p, rb)
    phc1, phc2 = _sc_gather(ph1, ph2, ix1, ix2)
    ctc1 = jnp.where(slot < cnt1, tgt[ix1] - _CUT0, -1)
    ctc2 = jnp.where(slot < cnt2, tgt[ix2] - _CUT1, -1)
    ct1b = jnp.broadcast_to(ctc1[:, None], (n, 128))
    ct2b = jnp.broadcast_to(ctc2[:, None], (n, 128))

    hw, hb, ncb_h = _pad_wb(jnp.concatenate([w0, cluster_weight], axis=0),
                            jnp.concatenate([b0, cluster_bias], axis=0), 1024)
    w1p, b1p, ncb_1 = _pad_wb(w1, b1, 2048)
    w2c = jnp.concatenate(
        [w2, jnp.zeros((w2.shape[0], 128 - w2.shape[1]), jnp.float32)],
        axis=1)
    w2p, b2p, ncb_2 = _pad_wb(w2c, b2, 2048)

    tgtb = jnp.broadcast_to(tgt[:, None], (n, 128))
    nfull = jnp.full((1,), n, jnp.int32)

    head = _cluster_nll(input, proj0, hw, hb, tgtb, nfull,
                        cb=1024, ncb=ncb_h, head=True, rb=rb)
    t1c = _cluster_nll(phc1, proj1, w1p, b1p, ct1b, cnt1.reshape(1),
                       cb=2048, ncb=ncb_1, head=False, rb=rb)
    t2c = _cluster_nll(phc2, proj2p, w2p, b2p, ct2b, cnt2.reshape(1),
                       cb=2048, ncb=ncb_2, head=False, rb=rb)

    nllt = jnp.broadcast_to(jnp.concatenate([t1c, t2c], axis=0), (2 * n, 128))
    g = _sc_nll_gather(nllt, pos)

    head2 = jnp.broadcast_to(head, (n, 16))
    clid2 = jnp.broadcast_to(clid[:, None], (n, 16))
    return _combine(head2, g[:, :16], clid2)[:, 0]
